# Initial kernel scaffold; baseline (speedup 1.0000x reference)
import jax
import jax.numpy as jnp
from jax.experimental import pallas as pl


def kernel(x, edge_index, batch, W1, b1, W2, b2, Wfc, bfc):
    raise NotImplementedError("write your pallas kernel here")


# speedup vs baseline: 4.5823x; 4.5823x over previous
import functools

import jax
import jax.numpy as jnp
from jax.experimental import pallas as pl
from jax.experimental.pallas import tpu as pltpu
from jax.experimental.pallas import tpu_sc as plsc

_HIGH = jax.lax.Precision.HIGHEST

_NUM_CORES = 2
_NUM_SUBCORES = 16
_EDGE_BLOCK = 512
_F = 128
_HALF = 5120
_SCRATCH = 128
_ACC = _HALF + _SCRATCH


def _segment_sum_sc(y, src3, dst3, zeros):
  n, f = y.shape
  nblk = src3.shape[0] // _NUM_SUBCORES
  blk = src3.shape[2]
  chunk = _ACC // _NUM_SUBCORES

  mesh = plsc.VectorSubcoreMesh(core_axis_name="c", subcore_axis_name="s")

  @functools.partial(
      pl.kernel,
      out_type=jax.ShapeDtypeStruct((_NUM_CORES * _ACC, f), jnp.float32),
      mesh=mesh,
      scratch_types=[
          pltpu.VMEM((1, blk), jnp.int32),
          pltpu.VMEM((1, blk), jnp.int32),
          pltpu.VMEM((blk, f), jnp.float32),
          pltpu.VMEM_SHARED((_ACC, f), jnp.float32),
          pltpu.SemaphoreType.DMA,
      ],
  )
  def seg_sum(y_hbm, src_hbm, dst_hbm, zero_hbm, out_hbm, src_v, dst_v,
              rows_v, acc, sem):
    cid = jax.lax.axis_index("c")
    sid = jax.lax.axis_index("s")
    pltpu.sync_copy(zero_hbm.at[pl.ds(sid * chunk, chunk)],
                    acc.at[pl.ds(sid * chunk, chunk)])
    plsc.subcore_barrier()

    @pl.loop(0, nblk)
    def _(bi):
      slab = sid * nblk + bi
      pltpu.sync_copy(src_hbm.at[slab], src_v)
      pltpu.sync_copy(dst_hbm.at[cid * (_NUM_SUBCORES * nblk) + slab], dst_v)
      pltpu.async_copy(y_hbm.at[src_v.at[0]], rows_v, sem).wait()
      pltpu.sync_copy(rows_v, acc.at[dst_v.at[0]], add=True)

    plsc.subcore_barrier()
    pltpu.sync_copy(acc.at[pl.ds(sid * chunk, chunk)],
                    out_hbm.at[pl.ds(cid * _ACC + sid * chunk, chunk)])

  return seg_sum(y, src3, dst3, zeros)


def _gather_rows(p_ref, n):
  return jnp.concatenate(
      [p_ref[:_HALF], p_ref[_ACC:_ACC + (n - _HALF)]], axis=0)


def _mm_body(x_ref, w_ref, o_ref):
  o_ref[:] = jnp.dot(x_ref[:], w_ref[:], precision=_HIGH)


def _layer2_body(n, y1_ref, p_ref, b1_ref, w2t_ref, o_ref):
  h = jnp.maximum(y1_ref[:] + _gather_rows(p_ref, n) + b1_ref[:], 0.0)
  o_ref[:] = jnp.dot(h, w2t_ref[:], precision=_HIGH)


def _final_body(n, g, y2_ref, q_ref, b2_ref, batch_ref, wfct_ref,
                bfc_ref, o_ref):
  h2 = jnp.maximum(y2_ref[:] + _gather_rows(q_ref, n) + b2_ref[:], 0.0)
  g_iota = jax.lax.broadcasted_iota(jnp.int32, (n, g), 1)
  onehot = (batch_ref[:] == g_iota).astype(jnp.float32)
  pooled = jax.lax.dot_general(
      onehot, h2, dimension_numbers=(((0,), (0,)), ((), ())),
      precision=_HIGH)
  logits = jnp.dot(pooled, wfct_ref[:], precision=_HIGH) + bfc_ref[:]
  m = jnp.max(logits, axis=1, keepdims=True)
  lse = m + jnp.log(jnp.sum(jnp.exp(logits - m), axis=1, keepdims=True))
  o_ref[:] = logits - lse


def _pad_cols(a, width):
  r, c = a.shape
  return jnp.concatenate([a, jnp.zeros((r, width - c), a.dtype)], axis=1)


@jax.jit
def kernel(x, edge_index, batch, W1, b1, W2, b2, Wfc, bfc):
  n, d = x.shape
  h = W1.shape[0]
  c = Wfc.shape[0]
  e = edge_index.shape[1]
  g = 128
  f = _F

  blk = _EDGE_BLOCK
  eps = e // _NUM_SUBCORES
  nblk = (eps + blk - 1) // blk
  eps_pad = nblk * blk
  pad = eps_pad - eps

  src = edge_index[0]
  dst = edge_index[1]
  spread = jnp.arange(pad * _NUM_SUBCORES, dtype=jnp.int32)
  src_w = jnp.concatenate(
      [src.reshape(_NUM_SUBCORES, eps),
       (spread % n).reshape(_NUM_SUBCORES, pad)], axis=1)
  src3 = src_w.reshape(_NUM_SUBCORES * nblk, 1, blk)

  scratch = _HALF + (jnp.arange(e, dtype=jnp.int32) % _SCRATCH)
  pad_scratch = _HALF + (spread % _SCRATCH).reshape(_NUM_SUBCORES, pad)
  dparts = []
  for cid in range(_NUM_CORES):
    lo = cid * _HALF
    local = jnp.where((dst >= lo) & (dst < lo + _HALF), dst - lo, scratch)
    dparts.append(jnp.concatenate(
        [local.reshape(_NUM_SUBCORES, eps), pad_scratch], axis=1))
  dst3 = jnp.stack(dparts).reshape(_NUM_CORES * _NUM_SUBCORES * nblk, 1, blk)

  zeros = jnp.zeros((_ACC, f), jnp.float32)
  batch_col = batch.reshape(n, 1)

  w1t = _pad_cols(W1.T, f)
  w2t = _pad_cols(jnp.concatenate(
      [W2.T, jnp.zeros((f - h, h), jnp.float32)], axis=0), f)
  b1p = _pad_cols(b1.reshape(1, h), f)
  b2p = _pad_cols(b2.reshape(1, h), f)
  wfct = jnp.concatenate(
      [Wfc.T, jnp.zeros((f - h, c), jnp.float32)], axis=0)

  y1 = pl.pallas_call(
      _mm_body,
      out_shape=jax.ShapeDtypeStruct((n, f), jnp.float32),
  )(x, w1t)

  p = _segment_sum_sc(y1, src3, dst3, zeros)

  y2 = pl.pallas_call(
      functools.partial(_layer2_body, n),
      out_shape=jax.ShapeDtypeStruct((n, f), jnp.float32),
  )(y1, p, b1p, w2t)

  q = _segment_sum_sc(y2, src3, dst3, zeros)

  out = pl.pallas_call(
      functools.partial(_final_body, n, g),
      out_shape=jax.ShapeDtypeStruct((g, c), jnp.float32),
  )(y2, q, b2p, batch_col, wfct, bfc.reshape(1, c))

  return out


# speedup vs baseline: 6.3351x; 1.3825x over previous
import functools

import jax
import jax.numpy as jnp
from jax.experimental import pallas as pl
from jax.experimental.pallas import tpu as pltpu
from jax.experimental.pallas import tpu_sc as plsc

_HIGH = jax.lax.Precision.HIGHEST

_NUM_CORES = 2
_NUM_SUBCORES = 16
_EDGE_BLOCK = 256
_F = 128
_HALF = 5120
_SCRATCH = 128
_ACC = _HALF + _SCRATCH
_NIDX = 8
_NROW = 2


def _segment_sum_sc(y, src3, dst3, zeros, nblk):
  n, f = y.shape
  blk = src3.shape[2]
  chunk = _ACC // _NUM_SUBCORES

  mesh = plsc.VectorSubcoreMesh(core_axis_name="c", subcore_axis_name="s")

  scratch = (
      [pltpu.VMEM((1, blk), jnp.int32) for _ in range(2 * _NIDX)]
      + [pltpu.VMEM((blk, f), jnp.float32) for _ in range(_NROW)]
      + [pltpu.VMEM_SHARED((_ACC, f), jnp.float32)]
      + [pltpu.SemaphoreType.DMA for _ in range(_NIDX + 2 * _NROW)]
  )

  @functools.partial(
      pl.kernel,
      out_type=jax.ShapeDtypeStruct((_NUM_CORES * _ACC, f), jnp.float32),
      mesh=mesh,
      scratch_types=scratch,
  )
  def seg_sum(y_hbm, src_hbm, dst_hbm, zero_hbm, out_hbm, *sc):
    src_v = sc[:_NIDX]
    dst_v = sc[_NIDX:2 * _NIDX]
    rows_v = sc[2 * _NIDX:2 * _NIDX + _NROW]
    acc = sc[2 * _NIDX + _NROW]
    sems = sc[2 * _NIDX + _NROW + 1:]
    isem = sems[:_NIDX]
    gsem = sems[_NIDX:_NIDX + _NROW]
    ssem = sems[_NIDX + _NROW:]

    cid = jax.lax.axis_index("c")
    sid = jax.lax.axis_index("s")
    pltpu.sync_copy(zero_hbm.at[pl.ds(sid * chunk, chunk)],
                    acc.at[pl.ds(sid * chunk, chunk)])
    plsc.subcore_barrier()

    sbase = sid * nblk
    dbase = cid * (_NUM_SUBCORES * nblk) + sid * nblk

    def idx_load(i, ib):
      pltpu.async_copy(src_hbm.at[sbase + i], src_v[ib], isem[ib])
      pltpu.async_copy(dst_hbm.at[dbase + i], dst_v[ib], isem[ib])

    def wait_idx(ib):
      pltpu.make_async_copy(src_hbm.at[0], src_v[ib], isem[ib]).wait()
      pltpu.make_async_copy(dst_hbm.at[0], dst_v[ib], isem[ib]).wait()

    def gather(ib, rb):
      pltpu.async_copy(y_hbm.at[src_v[ib].at[0]], rows_v[rb], gsem[rb])

    def wait_gather(rb):
      pltpu.make_async_copy(zero_hbm.at[pl.ds(0, blk)], rows_v[rb],
                            gsem[rb]).wait()

    def scatter(ib, rb):
      pltpu.async_copy(rows_v[rb], acc.at[dst_v[ib].at[0]], ssem[rb],
                       add=True)

    def wait_scatter(rb):
      pltpu.make_async_copy(zero_hbm.at[pl.ds(0, blk)], rows_v[rb],
                            ssem[rb]).wait()

    def body(i, b, first=False, last=False, load7=True):
      rb = b % _NROW
      rb2 = (b + 1) % _NROW
      ib = b
      ib2 = (b + 1) % _NIDX
      ib7 = (b + 7) % _NIDX
      wait_gather(rb)
      scatter(ib, rb)
      if not last:
        if not first:
          wait_scatter(rb2)
        wait_idx(ib2)
        gather(ib2, rb2)
        if load7:
          idx_load(i + 7, ib7)

    for j in range(_NIDX - 1):
      idx_load(j, j)
    wait_idx(0)
    gather(0, 0)

    for b in range(8):
      body(b, b, first=(b == 0))

    @pl.loop(8, nblk - 8, step=8)
    def _(g):
      for b in range(8):
        body(g + b, b)

    for b in range(8):
      i = nblk - 8 + b
      body(i, b, last=(i == nblk - 1), load7=(i + 7 < nblk))

    wait_scatter(0)
    wait_scatter(1)

    plsc.subcore_barrier()
    pltpu.sync_copy(acc.at[pl.ds(sid * chunk, chunk)],
                    out_hbm.at[pl.ds(cid * _ACC + sid * chunk, chunk)])

  return seg_sum(y, src3, dst3, zeros)


def _gather_rows(p_ref, n):
  return jnp.concatenate(
      [p_ref[:_HALF], p_ref[_ACC:_ACC + (n - _HALF)]], axis=0)


def _mm_body(x_ref, w_ref, o_ref):
  o_ref[:] = jnp.dot(x_ref[:], w_ref[:], precision=_HIGH)


def _layer2_body(n, y1_ref, p_ref, b1_ref, w2t_ref, o_ref):
  h = jnp.maximum(y1_ref[:] + _gather_rows(p_ref, n) + b1_ref[:], 0.0)
  o_ref[:] = jnp.dot(h, w2t_ref[:], precision=_HIGH)


def _final_body(n, g, y2_ref, q_ref, b2_ref, batch_ref, wfct_ref,
                bfc_ref, o_ref):
  h2 = jnp.maximum(y2_ref[:] + _gather_rows(q_ref, n) + b2_ref[:], 0.0)
  g_iota = jax.lax.broadcasted_iota(jnp.int32, (n, g), 1)
  onehot = (batch_ref[:] == g_iota).astype(jnp.float32)
  pooled = jax.lax.dot_general(
      onehot, h2, dimension_numbers=(((0,), (0,)), ((), ())),
      precision=_HIGH)
  logits = jnp.dot(pooled, wfct_ref[:], precision=_HIGH) + bfc_ref[:]
  m = jnp.max(logits, axis=1, keepdims=True)
  lse = m + jnp.log(jnp.sum(jnp.exp(logits - m), axis=1, keepdims=True))
  o_ref[:] = logits - lse


def _pad_cols(a, width):
  r, c = a.shape
  return jnp.concatenate([a, jnp.zeros((r, width - c), a.dtype)], axis=1)


@jax.jit
def kernel(x, edge_index, batch, W1, b1, W2, b2, Wfc, bfc):
  n, d = x.shape
  h = W1.shape[0]
  c = Wfc.shape[0]
  e = edge_index.shape[1]
  g = 128
  f = _F

  blk = _EDGE_BLOCK
  eps = e // _NUM_SUBCORES
  nblk = (eps + blk - 1) // blk
  nblk = ((nblk + 7) // 8) * 8
  eps_pad = nblk * blk
  pad = eps_pad - eps

  src = edge_index[0]
  dst = edge_index[1]
  spread = jnp.arange(pad * _NUM_SUBCORES, dtype=jnp.int32)
  src_w = jnp.concatenate(
      [src.reshape(_NUM_SUBCORES, eps),
       (spread % n).reshape(_NUM_SUBCORES, pad)], axis=1)
  src3 = src_w.reshape(_NUM_SUBCORES * nblk, 1, blk)

  scratch = _HALF + (jnp.arange(e, dtype=jnp.int32) % _SCRATCH)
  pad_scratch = _HALF + (spread % _SCRATCH).reshape(_NUM_SUBCORES, pad)
  dparts = []
  for cid in range(_NUM_CORES):
    lo = cid * _HALF
    local = jnp.where((dst >= lo) & (dst < lo + _HALF), dst - lo, scratch)
    dparts.append(jnp.concatenate(
        [local.reshape(_NUM_SUBCORES, eps), pad_scratch], axis=1))
  dst3 = jnp.stack(dparts).reshape(_NUM_CORES * _NUM_SUBCORES * nblk, 1, blk)

  zeros = jnp.zeros((_ACC, f), jnp.float32)
  batch_col = batch.reshape(n, 1)

  w1t = _pad_cols(W1.T, f)
  w2t = _pad_cols(jnp.concatenate(
      [W2.T, jnp.zeros((f - h, h), jnp.float32)], axis=0), f)
  b1p = _pad_cols(b1.reshape(1, h), f)
  b2p = _pad_cols(b2.reshape(1, h), f)
  wfct = jnp.concatenate(
      [Wfc.T, jnp.zeros((f - h, c), jnp.float32)], axis=0)

  y1 = pl.pallas_call(
      _mm_body,
      out_shape=jax.ShapeDtypeStruct((n, f), jnp.float32),
  )(x, w1t)

  p = _segment_sum_sc(y1, src3, dst3, zeros, nblk)

  y2 = pl.pallas_call(
      functools.partial(_layer2_body, n),
      out_shape=jax.ShapeDtypeStruct((n, f), jnp.float32),
  )(y1, p, b1p, w2t)

  q = _segment_sum_sc(y2, src3, dst3, zeros, nblk)

  out = pl.pallas_call(
      functools.partial(_final_body, n, g),
      out_shape=jax.ShapeDtypeStruct((g, c), jnp.float32),
  )(y2, q, b2p, batch_col, wfct, bfc.reshape(1, c))

  return out


# speedup vs baseline: 8.8944x; 1.4040x over previous
import functools

import jax
import jax.numpy as jnp
from jax.experimental import pallas as pl
from jax.experimental.pallas import tpu as pltpu
from jax.experimental.pallas import tpu_sc as plsc

_HIGH = jax.lax.Precision.HIGHEST

_NUM_CORES = 2
_NUM_SUBCORES = 16
_EDGE_BLOCK = 256
_F = 128
_HALFN = 5000
_SCRATCH = 120
_ACC = _HALFN + _SCRATCH
_NIDX = 8
_NROW = 2


def _segment_sum_sc(u, src3, dst3, zeros, nblk):
  n, f = u.shape
  blk = src3.shape[2]
  chunk = _ACC // _NUM_SUBCORES

  mesh = plsc.VectorSubcoreMesh(core_axis_name="c", subcore_axis_name="s")

  scratch = (
      [pltpu.VMEM((1, blk), jnp.int32) for _ in range(2 * _NIDX)]
      + [pltpu.VMEM((blk, f), jnp.float32) for _ in range(_NROW)]
      + [pltpu.VMEM_SHARED((_ACC, f), jnp.float32)]
      + [pltpu.SemaphoreType.DMA for _ in range(_NIDX + 2 * _NROW)]
  )

  @functools.partial(
      pl.kernel,
      out_type=jax.ShapeDtypeStruct((_NUM_CORES * _ACC, f), jnp.float32),
      mesh=mesh,
      scratch_types=scratch,
  )
  def seg_sum(u_hbm, src_hbm, dst_hbm, zero_hbm, out_hbm, *sc):
    src_v = sc[:_NIDX]
    dst_v = sc[_NIDX:2 * _NIDX]
    rows_v = sc[2 * _NIDX:2 * _NIDX + _NROW]
    acc = sc[2 * _NIDX + _NROW]
    sems = sc[2 * _NIDX + _NROW + 1:]
    isem = sems[:_NIDX]
    gsem = sems[_NIDX:_NIDX + _NROW]
    ssem = sems[_NIDX + _NROW:]

    cid = jax.lax.axis_index("c")
    sid = jax.lax.axis_index("s")
    pltpu.sync_copy(zero_hbm.at[pl.ds(sid * chunk, chunk)],
                    acc.at[pl.ds(sid * chunk, chunk)])
    plsc.subcore_barrier()

    base = cid * (_NUM_SUBCORES * nblk) + sid * nblk

    def idx_load(i, ib):
      pltpu.async_copy(src_hbm.at[base + i], src_v[ib], isem[ib])
      pltpu.async_copy(dst_hbm.at[base + i], dst_v[ib], isem[ib])

    def wait_idx(ib):
      pltpu.make_async_copy(src_hbm.at[0], src_v[ib], isem[ib]).wait()
      pltpu.make_async_copy(dst_hbm.at[0], dst_v[ib], isem[ib]).wait()

    def gather(ib, rb):
      pltpu.async_copy(u_hbm.at[src_v[ib].at[0]], rows_v[rb], gsem[rb])

    def wait_gather(rb):
      pltpu.make_async_copy(zero_hbm.at[pl.ds(0, blk)], rows_v[rb],
                            gsem[rb]).wait()

    def scatter(ib, rb):
      pltpu.async_copy(rows_v[rb], acc.at[dst_v[ib].at[0]], ssem[rb],
                       add=True)

    def wait_scatter(rb):
      pltpu.make_async_copy(zero_hbm.at[pl.ds(0, blk)], rows_v[rb],
                            ssem[rb]).wait()

    def body(i, b, first=False, last=False, load7=True):
      rb = b % _NROW
      rb2 = (b + 1) % _NROW
      ib = b
      ib2 = (b + 1) % _NIDX
      ib7 = (b + 7) % _NIDX
      wait_gather(rb)
      scatter(ib, rb)
      if not last:
        if not first:
          wait_scatter(rb2)
        wait_idx(ib2)
        gather(ib2, rb2)
        if load7:
          idx_load(i + 7, ib7)

    for j in range(_NIDX - 1):
      idx_load(j, j)
    wait_idx(0)
    gather(0, 0)

    for b in range(8):
      body(b, b, first=(b == 0))

    @pl.loop(8, nblk - 8, step=8)
    def _(g):
      for b in range(8):
        body(g + b, b)

    for b in range(8):
      i = nblk - 8 + b
      body(i, b, last=(i == nblk - 1), load7=(i + 7 < nblk))

    wait_scatter(0)
    wait_scatter(1)

    plsc.subcore_barrier()
    pltpu.sync_copy(acc.at[pl.ds(sid * chunk, chunk)],
                    out_hbm.at[pl.ds(cid * _ACC + sid * chunk, chunk)])

  return seg_sum(u, src3, dst3, zeros)


def _layer1_body(x2_ref, k1_ref, k2_ref, k3_ref, k4_ref, o_ref):
  for q, k_ref in enumerate((k1_ref, k2_ref, k3_ref, k4_ref)):
    o_ref[q * _HALFN:(q + 1) * _HALFN] = jnp.dot(
        x2_ref[:], k_ref[:], precision=_HIGH)


def _layer2_body(u1_ref, p_ref, b1d_ref, k1_ref, k2_ref, k3_ref, k4_ref,
                 o_ref):
  y1p = u1_ref[0:_HALFN] + u1_ref[3 * _HALFN:4 * _HALFN]
  aggp = p_ref[0:_HALFN] + p_ref[_ACC:_ACC + _HALFN]
  h = jnp.maximum(y1p + aggp + b1d_ref[:], 0.0)
  for q, k_ref in enumerate((k1_ref, k2_ref, k3_ref, k4_ref)):
    o_ref[q * _HALFN:(q + 1) * _HALFN] = jnp.dot(
        h, k_ref[:], precision=_HIGH)


def _final_body(g, u2_ref, q_ref, b2d_ref, slo_ref, shi_ref, be_ref,
                bo_ref, wfct_ref, bfc_ref, o_ref):
  y2p = u2_ref[0:_HALFN] + u2_ref[3 * _HALFN:4 * _HALFN]
  aggp = q_ref[0:_HALFN] + q_ref[_ACC:_ACC + _HALFN]
  h2 = jnp.maximum(y2p + aggp + b2d_ref[:], 0.0)
  lo = jnp.dot(h2, slo_ref[:], precision=_HIGH)
  hi = jnp.dot(h2, shi_ref[:], precision=_HIGH)
  g_iota = jax.lax.broadcasted_iota(jnp.int32, (_HALFN, g), 1)
  oh_e = (be_ref[:] == g_iota).astype(jnp.float32)
  oh_o = (bo_ref[:] == g_iota).astype(jnp.float32)
  pooled = (
      jax.lax.dot_general(oh_e, lo,
                          dimension_numbers=(((0,), (0,)), ((), ())),
                          precision=_HIGH)
      + jax.lax.dot_general(oh_o, hi,
                            dimension_numbers=(((0,), (0,)), ((), ())),
                            precision=_HIGH))
  logits = jnp.dot(pooled, wfct_ref[:], precision=_HIGH) + bfc_ref[:]
  m = jnp.max(logits, axis=1, keepdims=True)
  lse = m + jnp.log(jnp.sum(jnp.exp(logits - m), axis=1, keepdims=True))
  o_ref[:] = logits - lse


def _quad_weights(wt):
  i_dim, h = wt.shape
  z = jnp.zeros((i_dim, 128), jnp.float32)
  half = i_dim // 2
  k1 = z.at[:half, :h].set(wt[:half])
  k2 = z.at[half:, :h].set(wt[half:])
  k3 = z.at[:half, h:2 * h].set(wt[:half])
  k4 = z.at[half:, h:2 * h].set(wt[half:])
  return k1, k2, k3, k4


@jax.jit
def kernel(x, edge_index, batch, W1, b1, W2, b2, Wfc, bfc):
  n, d = x.shape
  h = W1.shape[0]
  c = Wfc.shape[0]
  e = edge_index.shape[1]
  g = 128

  blk = _EDGE_BLOCK
  nsub = _NUM_CORES * _NUM_SUBCORES
  eps = e // nsub
  nblk = (eps + blk - 1) // blk
  nblk = ((nblk + 7) // 8) * 8
  eps_pad = nblk * blk
  pad = eps_pad - eps

  src = edge_index[0]
  dst = edge_index[1]
  gsrc = (src // 2) + _HALFN * (src % 2) + 2 * _HALFN * (dst % 2)
  gdst = dst // 2
  spread = jnp.arange(pad * nsub, dtype=jnp.int32)
  pad_src = (spread % (4 * _HALFN)).reshape(nsub, pad)
  pad_dst = (_HALFN + spread % _SCRATCH).reshape(nsub, pad)
  src3 = jnp.concatenate(
      [gsrc.reshape(nsub, eps), pad_src], axis=1).reshape(
          nsub * nblk, 1, blk)
  dst3 = jnp.concatenate(
      [gdst.reshape(nsub, eps), pad_dst], axis=1).reshape(
          nsub * nblk, 1, blk)

  zeros = jnp.zeros((_ACC, _F), jnp.float32)

  x2 = x.reshape(n // 2, 2 * d)
  batch2 = batch.reshape(n // 2, 2)
  be = batch2[:, 0:1]
  bo = batch2[:, 1:2]

  w1t = W1.T
  w1_dual = jnp.concatenate([w1t, w1t], axis=0)
  k1s = _quad_weights(w1_dual)

  w2t = W2.T
  k2s = _quad_weights(jnp.concatenate([w2t, w2t], axis=0))

  b1d = jnp.concatenate([b1, b1]).reshape(1, 2 * h)
  b2d = jnp.concatenate([b2, b2]).reshape(1, 2 * h)
  eye = jnp.eye(h, dtype=jnp.float32)
  zed = jnp.zeros((h, h), jnp.float32)
  slo = jnp.concatenate([eye, zed], axis=0)
  shi = jnp.concatenate([zed, eye], axis=0)
  wfct = Wfc.T

  u1 = pl.pallas_call(
      _layer1_body,
      out_shape=jax.ShapeDtypeStruct((4 * _HALFN, _F), jnp.float32),
  )(x2, *k1s)

  p = _segment_sum_sc(u1, src3, dst3, zeros, nblk)

  u2 = pl.pallas_call(
      _layer2_body,
      out_shape=jax.ShapeDtypeStruct((4 * _HALFN, _F), jnp.float32),
  )(u1, p, b1d, *k2s)

  q = _segment_sum_sc(u2, src3, dst3, zeros, nblk)

  out = pl.pallas_call(
      functools.partial(_final_body, g),
      out_shape=jax.ShapeDtypeStruct((g, c), jnp.float32),
  )(u2, q, b2d, slo, shi, be, bo, wfct, bfc.reshape(1, c))

  return out
